# baseline (speedup 1.0000x reference)
import functools

import jax
import jax.numpy as jnp
from jax import lax
from jax.experimental import pallas as pl
from jax.experimental.pallas import tpu as pltpu
from jax.experimental.pallas import tpu_sc as plsc

N = 50000
E = 1600000
D = 31
L = 5
DP = 32
HF = 16

NS = 16
BATCH = 256
ROWS_PER_TILE = 400
E_PAD = NS * ROWS_PER_TILE * BATCH
CHUNK = 80
N_CHUNKS = ROWS_PER_TILE // CHUNK
NB = 8
N_TILE = 3128
N_PAD = NS * N_TILE


def _sc_aggregate_body(h_hbm, idx_hbm, z_hbm, out_hbm, src_v, dst_v, agg_s,
                       *bs):
    bufs, sems = bs[:NB], bs[NB:]
    c = lax.axis_index("c")
    s = lax.axis_index("s")

    pltpu.sync_copy(z_hbm, agg_s.at[pl.ds(s * N_TILE, N_TILE)])
    plsc.subcore_barrier()

    def fire(i, q):
        pltpu.async_copy(h_hbm.at[src_v.at[q]], bufs[i], sems[i])

    def drain(i, q):
        pltpu.make_async_copy(h_hbm.at[src_v.at[q]], bufs[i], sems[i]).wait()
        pltpu.sync_copy(bufs[i], agg_s.at[dst_v.at[q]], add=True)

    for k in range(N_CHUNKS):
        pltpu.sync_copy(idx_hbm.at[2 * c, s, pl.ds(k * CHUNK, CHUNK)], src_v)
        pltpu.sync_copy(idx_hbm.at[1, s, pl.ds(k * CHUNK, CHUNK)], dst_v)

        for i in range(NB):
            fire(i, i)

        def step(it, carry):
            base = NB * it
            for i in range(NB):
                drain(i, base + i)
                fire(i, base + NB + i)
            return carry

        lax.fori_loop(0, CHUNK // NB - 1, step, 0)
        for i in range(NB):
            drain(i, CHUNK - NB + i)

    plsc.subcore_barrier()
    pltpu.sync_copy(agg_s.at[pl.ds(s * N_TILE, N_TILE)],
                    out_hbm.at[c, pl.ds(s * N_TILE, N_TILE)])


_sc_aggregate = functools.partial(
    pl.kernel,
    out_type=jax.ShapeDtypeStruct((2, N_PAD, HF), jnp.float32),
    mesh=plsc.VectorSubcoreMesh(core_axis_name="c", subcore_axis_name="s"),
    scratch_types=[
        pltpu.VMEM((CHUNK, BATCH), jnp.int32),
        pltpu.VMEM((CHUNK, BATCH), jnp.int32),
        pltpu.VMEM_SHARED((N_PAD, HF), jnp.float32),
    ] + [pltpu.VMEM((BATCH, HF), jnp.float32) for _ in range(NB)]
      + [pltpu.SemaphoreType.DMA for _ in range(NB)],
    compiler_params=pltpu.CompilerParams(use_tc_tiling_on_sc=False),
)(_sc_aggregate_body)


def _tc_dense_body(h_ref, a_ref, ws_ref, wn_ref, b_ref, out_ref):
    a0 = a_ref[0]
    a1 = a_ref[1]
    deg = a1[:, HF - 1:HF]
    inv = 1.0 / jnp.maximum(deg, 1.0)
    dot = functools.partial(jnp.dot, preferred_element_type=jnp.float32)
    pre = (dot(h_ref[0], ws_ref[:HF]) + dot(h_ref[1], ws_ref[HF:])
           + dot(a0 * inv, wn_ref[:HF]) + dot(a1 * inv, wn_ref[HF:])
           + b_ref[...])
    r = jnp.maximum(pre, 0.0)
    out_ref[0] = r[:, :HF]
    out_ref[1] = r[:, HF:]


_BLK = 5000


def _tc_dense(h2, agg, ws, wn, b2):
    return pl.pallas_call(
        _tc_dense_body,
        grid=(N // _BLK,),
        in_specs=[
            pl.BlockSpec((2, _BLK, HF), lambda i: (0, i, 0)),
            pl.BlockSpec((2, _BLK, HF), lambda i: (0, i, 0)),
            pl.BlockSpec((DP, DP), lambda i: (0, 0)),
            pl.BlockSpec((DP, DP), lambda i: (0, 0)),
            pl.BlockSpec((1, DP), lambda i: (0, 0)),
        ],
        out_specs=pl.BlockSpec((2, _BLK, HF), lambda i: (0, i, 0)),
        out_shape=jax.ShapeDtypeStruct((2, N, HF), jnp.float32),
    )(h2, agg, ws, wn, b2)


def kernel(x, edge_index, W_self, W_neigh, b):
    src = edge_index[0].astype(jnp.int32)
    dst = edge_index[1].astype(jnp.int32)

    pad = E_PAD - E
    fill = jnp.arange(pad, dtype=jnp.int32)
    src_p = jnp.concatenate([src, (fill * 97) % N])
    dst_p = jnp.concatenate([dst, N + fill % (N_PAD - N)])
    idx = jnp.stack([src_p, dst_p, src_p + N]).reshape(
        3, NS, ROWS_PER_TILE, BATCH)

    h2 = jnp.stack([
        x[:, :HF],
        jnp.concatenate([x[:, HF:], jnp.ones((N, 1), jnp.float32)], axis=1),
    ])

    ws_p = jnp.pad(W_self, ((0, 0), (0, 1), (0, 1)))
    wn_p = jnp.pad(W_neigh, ((0, 0), (0, 1), (0, 1)))
    b_p = jnp.pad(b, ((0, 0), (0, 1))).at[:, D].set(1.0)

    z = jnp.zeros((N_TILE, HF), jnp.float32)

    def layer(h2c, wsb):
        ws_i, wn_i, b_i = wsb
        agg = _sc_aggregate(h2c.reshape(2 * N, HF), idx, z)
        return _tc_dense(h2c, agg, ws_i, wn_i, b_i), None

    h2, _ = lax.scan(layer, h2, (ws_p, wn_p, b_p[:, None, :]))
    return jnp.concatenate([h2[0], h2[1][:, :D - HF]], axis=1)


# speedup vs baseline: 1.0922x; 1.0922x over previous
import functools

import jax
import jax.numpy as jnp
from jax import lax
from jax.experimental import pallas as pl
from jax.experimental.pallas import tpu as pltpu
from jax.experimental.pallas import tpu_sc as plsc

N = 50000
E = 1600000
D = 31
L = 5
DP = 32
HF = 16

NS = 16
BATCH = 128
ROWS_PER_TILE = 800
E_PAD = NS * ROWS_PER_TILE * BATCH
CHUNK = 200
N_CHUNKS = ROWS_PER_TILE // CHUNK
NB = 10
N_TILE = 3128
N_PAD = NS * N_TILE


def _sc_aggregate_body(h_hbm, idx_hbm, z_hbm, out_hbm, src_v, dst_v, agg_s,
                       *bs):
    bufs, sems = bs[:NB], bs[NB:]
    c = lax.axis_index("c")
    s = lax.axis_index("s")

    pltpu.sync_copy(z_hbm, agg_s.at[pl.ds(s * N_TILE, N_TILE)])
    plsc.subcore_barrier()

    def fire(i, q):
        pltpu.async_copy(h_hbm.at[src_v.at[q]], bufs[i], sems[i])

    def drain(i, q):
        pltpu.make_async_copy(h_hbm.at[src_v.at[q]], bufs[i], sems[i]).wait()
        pltpu.sync_copy(bufs[i], agg_s.at[dst_v.at[q]], add=True)

    for k in range(N_CHUNKS):
        pltpu.sync_copy(idx_hbm.at[2 * c, s, pl.ds(k * CHUNK, CHUNK)], src_v)
        pltpu.sync_copy(idx_hbm.at[1, s, pl.ds(k * CHUNK, CHUNK)], dst_v)

        for i in range(NB):
            fire(i, i)

        def step(it, carry):
            base = NB * it
            for i in range(NB):
                drain(i, base + i)
                fire(i, base + NB + i)
            return carry

        lax.fori_loop(0, CHUNK // NB - 1, step, 0)
        for i in range(NB):
            drain(i, CHUNK - NB + i)

    plsc.subcore_barrier()
    pltpu.sync_copy(agg_s.at[pl.ds(s * N_TILE, N_TILE)],
                    out_hbm.at[c, pl.ds(s * N_TILE, N_TILE)])


_sc_aggregate = functools.partial(
    pl.kernel,
    out_type=jax.ShapeDtypeStruct((2, N_PAD, HF), jnp.float32),
    mesh=plsc.VectorSubcoreMesh(core_axis_name="c", subcore_axis_name="s"),
    scratch_types=[
        pltpu.VMEM((CHUNK, BATCH), jnp.int32),
        pltpu.VMEM((CHUNK, BATCH), jnp.int32),
        pltpu.VMEM_SHARED((N_PAD, HF), jnp.float32),
    ] + [pltpu.VMEM((BATCH, HF), jnp.float32) for _ in range(NB)]
      + [pltpu.SemaphoreType.DMA for _ in range(NB)],
    compiler_params=pltpu.CompilerParams(use_tc_tiling_on_sc=False),
)(_sc_aggregate_body)


def _tc_dense_body(h_ref, a_ref, ws_ref, wn_ref, b_ref, out_ref):
    a0 = a_ref[0]
    a1 = a_ref[1]
    deg = a1[:, HF - 1:HF]
    inv = 1.0 / jnp.maximum(deg, 1.0)
    dot = functools.partial(jnp.dot, preferred_element_type=jnp.float32)
    pre = (dot(h_ref[0], ws_ref[:HF]) + dot(h_ref[1], ws_ref[HF:])
           + dot(a0 * inv, wn_ref[:HF]) + dot(a1 * inv, wn_ref[HF:])
           + b_ref[...])
    r = jnp.maximum(pre, 0.0)
    out_ref[0] = r[:, :HF]
    out_ref[1] = r[:, HF:]


_BLK = 5000


def _tc_dense(h2, agg, ws, wn, b2):
    return pl.pallas_call(
        _tc_dense_body,
        grid=(N // _BLK,),
        in_specs=[
            pl.BlockSpec((2, _BLK, HF), lambda i: (0, i, 0)),
            pl.BlockSpec((2, _BLK, HF), lambda i: (0, i, 0)),
            pl.BlockSpec((DP, DP), lambda i: (0, 0)),
            pl.BlockSpec((DP, DP), lambda i: (0, 0)),
            pl.BlockSpec((1, DP), lambda i: (0, 0)),
        ],
        out_specs=pl.BlockSpec((2, _BLK, HF), lambda i: (0, i, 0)),
        out_shape=jax.ShapeDtypeStruct((2, N, HF), jnp.float32),
    )(h2, agg, ws, wn, b2)


def kernel(x, edge_index, W_self, W_neigh, b):
    src = edge_index[0].astype(jnp.int32)
    dst = edge_index[1].astype(jnp.int32)

    pad = E_PAD - E
    fill = jnp.arange(pad, dtype=jnp.int32)
    src_p = jnp.concatenate([src, (fill * 97) % N])
    dst_p = jnp.concatenate([dst, N + fill % (N_PAD - N)])
    idx = jnp.stack([src_p, dst_p, src_p + N]).reshape(
        3, NS, ROWS_PER_TILE, BATCH)

    h2 = jnp.stack([
        x[:, :HF],
        jnp.concatenate([x[:, HF:], jnp.ones((N, 1), jnp.float32)], axis=1),
    ])

    ws_p = jnp.pad(W_self, ((0, 0), (0, 1), (0, 1)))
    wn_p = jnp.pad(W_neigh, ((0, 0), (0, 1), (0, 1)))
    b_p = jnp.pad(b, ((0, 0), (0, 1))).at[:, D].set(1.0)

    z = jnp.zeros((N_TILE, HF), jnp.float32)

    def layer(h2c, wsb):
        ws_i, wn_i, b_i = wsb
        agg = _sc_aggregate(h2c.reshape(2 * N, HF), idx, z)
        return _tc_dense(h2c, agg, ws_i, wn_i, b_i), None

    h2, _ = lax.scan(layer, h2, (ws_p, wn_p, b_p[:, None, :]))
    return jnp.concatenate([h2[0], h2[1][:, :D - HF]], axis=1)


# speedup vs baseline: 1.0941x; 1.0017x over previous
import functools

import jax
import jax.numpy as jnp
from jax import lax
from jax.experimental import pallas as pl
from jax.experimental.pallas import tpu as pltpu
from jax.experimental.pallas import tpu_sc as plsc

N = 50000
E = 1600000
D = 31
L = 5
DP = 32
HF = 16

NS = 16
BATCH = 128
ROWS_PER_TILE = 800
E_PAD = NS * ROWS_PER_TILE * BATCH
CHUNK = 200
N_CHUNKS = ROWS_PER_TILE // CHUNK
NB = 10
N_TILE = 3128
N_PAD = NS * N_TILE


def _sc_aggregate_body(h_hbm, idx_hbm, z_hbm, out_hbm, src_v, dst_v, agg_s,
                       *bs):
    bufs, sems = bs[:NB], bs[NB:]
    c = lax.axis_index("c")
    s = lax.axis_index("s")

    pltpu.sync_copy(z_hbm, agg_s.at[pl.ds(s * N_TILE, N_TILE)])
    plsc.subcore_barrier()

    ssems = sems[NB:]

    def fire(i, q):
        pltpu.async_copy(h_hbm.at[src_v.at[q]], bufs[i], sems[i])

    def wait_gather(i, q):
        pltpu.make_async_copy(h_hbm.at[src_v.at[q]], bufs[i], sems[i]).wait()

    def fire_scatter(i, q):
        pltpu.async_copy(bufs[i], agg_s.at[dst_v.at[q]], ssems[i], add=True)

    def wait_scatter(i, q):
        pltpu.make_async_copy(bufs[i], agg_s.at[dst_v.at[q]],
                              ssems[i]).wait()

    def drain(i, q):
        wait_gather(i, q)
        pltpu.sync_copy(bufs[i], agg_s.at[dst_v.at[q]], add=True)

    for k in range(N_CHUNKS):
        pltpu.sync_copy(idx_hbm.at[2 * c, s, pl.ds(k * CHUNK, CHUNK)], src_v)
        pltpu.sync_copy(idx_hbm.at[1, s, pl.ds(k * CHUNK, CHUNK)], dst_v)

        for i in range(NB):
            fire(i, i)

        def step(it, carry):
            base = NB * it
            for g in range(2):
                lo, hi = g * NB // 2, (g + 1) * NB // 2
                for i in range(lo, hi):
                    wait_gather(i, base + i)
                    fire_scatter(i, base + i)
                for i in range(lo, hi):
                    wait_scatter(i, base + i)
                    fire(i, base + NB + i)
            return carry

        lax.fori_loop(0, CHUNK // NB - 1, step, 0)
        for i in range(NB):
            drain(i, CHUNK - NB + i)

    plsc.subcore_barrier()
    pltpu.sync_copy(agg_s.at[pl.ds(s * N_TILE, N_TILE)],
                    out_hbm.at[c, pl.ds(s * N_TILE, N_TILE)])


_sc_aggregate = functools.partial(
    pl.kernel,
    out_type=jax.ShapeDtypeStruct((2, N_PAD, HF), jnp.float32),
    mesh=plsc.VectorSubcoreMesh(core_axis_name="c", subcore_axis_name="s"),
    scratch_types=[
        pltpu.VMEM((CHUNK, BATCH), jnp.int32),
        pltpu.VMEM((CHUNK, BATCH), jnp.int32),
        pltpu.VMEM_SHARED((N_PAD, HF), jnp.float32),
    ] + [pltpu.VMEM((BATCH, HF), jnp.float32) for _ in range(NB)]
      + [pltpu.SemaphoreType.DMA for _ in range(2 * NB)],
    compiler_params=pltpu.CompilerParams(use_tc_tiling_on_sc=False),
)(_sc_aggregate_body)


def _tc_dense_body(h_ref, a_ref, ws_ref, wn_ref, b_ref, out_ref):
    a0 = a_ref[0]
    a1 = a_ref[1]
    deg = a1[:, HF - 1:HF]
    inv = 1.0 / jnp.maximum(deg, 1.0)
    dot = functools.partial(jnp.dot, preferred_element_type=jnp.float32)
    pre = (dot(h_ref[0], ws_ref[:HF]) + dot(h_ref[1], ws_ref[HF:])
           + dot(a0 * inv, wn_ref[:HF]) + dot(a1 * inv, wn_ref[HF:])
           + b_ref[...])
    r = jnp.maximum(pre, 0.0)
    out_ref[0] = r[:, :HF]
    out_ref[1] = r[:, HF:]


_BLK = 5000


def _tc_dense(h2, agg, ws, wn, b2):
    return pl.pallas_call(
        _tc_dense_body,
        grid=(N // _BLK,),
        in_specs=[
            pl.BlockSpec((2, _BLK, HF), lambda i: (0, i, 0)),
            pl.BlockSpec((2, _BLK, HF), lambda i: (0, i, 0)),
            pl.BlockSpec((DP, DP), lambda i: (0, 0)),
            pl.BlockSpec((DP, DP), lambda i: (0, 0)),
            pl.BlockSpec((1, DP), lambda i: (0, 0)),
        ],
        out_specs=pl.BlockSpec((2, _BLK, HF), lambda i: (0, i, 0)),
        out_shape=jax.ShapeDtypeStruct((2, N, HF), jnp.float32),
    )(h2, agg, ws, wn, b2)


def kernel(x, edge_index, W_self, W_neigh, b):
    src = edge_index[0].astype(jnp.int32)
    dst = edge_index[1].astype(jnp.int32)

    pad = E_PAD - E
    fill = jnp.arange(pad, dtype=jnp.int32)
    src_p = jnp.concatenate([src, (fill * 97) % N])
    dst_p = jnp.concatenate([dst, N + fill % (N_PAD - N)])
    idx = jnp.stack([src_p, dst_p, src_p + N]).reshape(
        3, NS, ROWS_PER_TILE, BATCH)

    h2 = jnp.stack([
        x[:, :HF],
        jnp.concatenate([x[:, HF:], jnp.ones((N, 1), jnp.float32)], axis=1),
    ])

    ws_p = jnp.pad(W_self, ((0, 0), (0, 1), (0, 1)))
    wn_p = jnp.pad(W_neigh, ((0, 0), (0, 1), (0, 1)))
    b_p = jnp.pad(b, ((0, 0), (0, 1))).at[:, D].set(1.0)

    z = jnp.zeros((N_TILE, HF), jnp.float32)

    def layer(h2c, wsb):
        ws_i, wn_i, b_i = wsb
        agg = _sc_aggregate(h2c.reshape(2 * N, HF), idx, z)
        return _tc_dense(h2c, agg, ws_i, wn_i, b_i), None

    h2, _ = lax.scan(layer, h2, (ws_p, wn_p, b_p[:, None, :]))
    return jnp.concatenate([h2[0], h2[1][:, :D - HF]], axis=1)
